# baseline (speedup 1.0000x reference)
import functools

import jax
import jax.numpy as jnp
from jax.experimental import pallas as pl

_NT = (((1,), (1,)), ((), ()))


def _pre_kernel(ctx_ref, cm_ref, w_ref, cn_ref, m_ref, w16_ref):
    c = ctx_ref[...]
    norms = jnp.sqrt(jnp.sum(c * c, axis=1, keepdims=True))
    cn_ref[...] = (c / jnp.clip(norms, 1e-8, None)).astype(jnp.bfloat16)
    seg = jax.lax.dot_general(cm_ref[...], c, _NT,
                              preferred_element_type=jnp.float32)
    m_ref[...] = jnp.max(seg, axis=0, keepdims=True)
    w16_ref[...] = w_ref[...].astype(jnp.bfloat16)


def _main_kernel(x_ref, cn_ref, m_ref, w16_ref, b_ref, out_ref, *, n_total):
    tb = out_ref.shape[0]
    l_dim = x_ref.shape[2]
    xb16 = x_ref[...].reshape(n_total * tb, l_dim).astype(jnp.bfloat16)
    s = jax.lax.dot_general(xb16, cn_ref[...], _NT,
                            preferred_element_type=jnp.float32)
    rowmax = jnp.max(s, axis=1, keepdims=True)
    mval = jnp.max(jnp.where(s == rowmax, m_ref[...], -jnp.inf),
                   axis=1, keepdims=True)
    act = jax.nn.sigmoid(mval) * (0.5 / n_total)
    h = jax.lax.dot_general(xb16, w16_ref[...], _NT,
                            preferred_element_type=jnp.float32) + b_ref[...]
    g = h * (1.0 + jax.lax.erf(h * 0.7071067811865476))
    contrib = g * act
    parts = [contrib[i * tb:(i + 1) * tb, :] for i in range(n_total)]
    while len(parts) > 1:
        parts = [parts[i] + parts[i + 1] for i in range(0, len(parts), 2)] + \
            (parts[-1:] if len(parts) % 2 else [])
    out_ref[...] = parts[0]


def kernel(x, ctx_mod, context, W, b):
    N, B, L = x.shape
    n_ctx = context.shape[0]

    cn16, m, w16 = pl.pallas_call(
        _pre_kernel,
        out_shape=[
            jax.ShapeDtypeStruct((n_ctx, L), jnp.bfloat16),
            jax.ShapeDtypeStruct((1, n_ctx), jnp.float32),
            jax.ShapeDtypeStruct((L, L), jnp.bfloat16),
        ],
    )(context, ctx_mod, W)

    b2 = b.reshape(1, L)

    tb = 256 if B % 256 == 0 else B
    out = pl.pallas_call(
        functools.partial(_main_kernel, n_total=N),
        grid=(B // tb,),
        in_specs=[
            pl.BlockSpec((N, tb, L), lambda bi: (0, bi, 0)),
            pl.BlockSpec((n_ctx, L), lambda bi: (0, 0)),
            pl.BlockSpec((1, n_ctx), lambda bi: (0, 0)),
            pl.BlockSpec((L, L), lambda bi: (0, 0)),
            pl.BlockSpec((1, L), lambda bi: (0, 0)),
        ],
        out_specs=pl.BlockSpec((tb, L), lambda bi: (bi, 0)),
        out_shape=jax.ShapeDtypeStruct((B, L), jnp.float32),
    )(x, cn16, m, w16, b2)
    return out


# speedup vs baseline: 1.0774x; 1.0774x over previous
import functools

import jax
import jax.numpy as jnp
from jax.experimental import pallas as pl
from jax.experimental.pallas import tpu as pltpu

_NT = (((1,), (1,)), ((), ()))


def _kernel(x_ref, ctx_ref, cm_ref, w_ref, b_ref, out_ref,
            cn16, m, w16, *, n_total):
    k = pl.program_id(0)

    @pl.when(k == 0)
    def _():
        c = ctx_ref[...]
        norms = jnp.sqrt(jnp.sum(c * c, axis=1, keepdims=True))
        cn16[...] = (c / jnp.clip(norms, 1e-8, None)).astype(jnp.bfloat16)
        seg = jax.lax.dot_general(cm_ref[...], c, _NT,
                                  preferred_element_type=jnp.float32)
        m[...] = jnp.max(seg, axis=0, keepdims=True)
        w16[...] = w_ref[...].astype(jnp.bfloat16)

    @pl.when(k > 0)
    def _():
        tb = out_ref.shape[0]
        l_dim = x_ref.shape[2]
        xb16 = x_ref[...].reshape(n_total * tb, l_dim).astype(jnp.bfloat16)
        s = jax.lax.dot_general(xb16, cn16[...], _NT,
                                preferred_element_type=jnp.float32)
        rowmax = jnp.max(s, axis=1, keepdims=True)
        mval = jnp.max(jnp.where(s == rowmax, m[...], -jnp.inf),
                       axis=1, keepdims=True)
        act = jax.nn.sigmoid(mval) * (0.5 / n_total)
        h = jax.lax.dot_general(xb16, w16[...], _NT,
                                preferred_element_type=jnp.float32) + b_ref[...]
        g = h * (1.0 + jax.lax.erf(h * 0.7071067811865476))
        contrib = g * act
        parts = [contrib[i * tb:(i + 1) * tb, :] for i in range(n_total)]
        while len(parts) > 1:
            parts = [parts[i] + parts[i + 1]
                     for i in range(0, len(parts), 2)] + \
                (parts[-1:] if len(parts) % 2 else [])
        out_ref[...] = parts[0]


def kernel(x, ctx_mod, context, W, b):
    N, B, L = x.shape
    n_ctx = context.shape[0]
    n_seg = ctx_mod.shape[0]
    b2 = b.reshape(1, L)

    tb = 128 if B % 128 == 0 else B
    out = pl.pallas_call(
        functools.partial(_kernel, n_total=N),
        grid=(B // tb + 1,),
        in_specs=[
            pl.BlockSpec((N, tb, L), lambda k: (0, jnp.maximum(k - 1, 0), 0)),
            pl.BlockSpec((n_ctx, L), lambda k: (0, 0)),
            pl.BlockSpec((n_seg, L), lambda k: (0, 0)),
            pl.BlockSpec((L, L), lambda k: (0, 0)),
            pl.BlockSpec((1, L), lambda k: (0, 0)),
        ],
        out_specs=pl.BlockSpec((tb, L), lambda k: (jnp.maximum(k - 1, 0), 0)),
        out_shape=jax.ShapeDtypeStruct((B, L), jnp.float32),
        scratch_shapes=[
            pltpu.VMEM((n_ctx, L), jnp.bfloat16),
            pltpu.VMEM((1, n_ctx), jnp.float32),
            pltpu.VMEM((L, L), jnp.bfloat16),
        ],
    )(x, context, ctx_mod, W, b2)
    return out
